# baseline (speedup 1.0000x reference)
import jax
import jax.numpy as jnp
from jax.experimental import pallas as pl
from jax.experimental.pallas import tpu as pltpu


def _fused_kernel(x_ref, w_ref, b_ref, a_ref, o_ref, xs_ref, s_ref):
    t = pl.program_id(1)
    cout = o_ref.shape[1]
    th = o_ref.shape[2] // 2
    w_out = o_ref.shape[3] // 2
    hh = x_ref.shape[2]

    @pl.when(t == 0)
    def _fill_input_scratch():
        xs_ref[...] = jnp.zeros_like(xs_ref)
        xs_ref[1:hh + 1, 1:w_out + 1, :] = jnp.transpose(x_ref[0], (1, 2, 0))

    row0 = pl.multiple_of(t * th, th)

    slabs = []
    for dy in range(3):
        rows = xs_ref[pl.ds(row0 + dy, th), :, :]
        for dx in range(3):
            slabs.append(rows[:, dx:dx + w_out, :])
    patch = jnp.concatenate(slabs, axis=-1).astype(jnp.bfloat16)
    kk = patch.shape[-1]

    acc = jnp.dot(patch.reshape(th * w_out, kk), w_ref[...],
                  preferred_element_type=jnp.float32)
    acc = acc + b_ref[0]
    alpha = a_ref[0]
    acc = jnp.where(acc >= 0.0, acc, alpha * acc)

    for i in range(2):
        for j in range(2):
            lane0 = i * 2 * cout + j * cout
            v = acc[:, lane0:lane0 + cout].reshape(th, w_out, cout)
            s_ref[pl.ds(i, th, 2), pl.ds(j, w_out, 2), :] = v

    o_ref[0] = jnp.transpose(s_ref[...], (2, 0, 1))


def kernel(x_nchw, weight, bias, alpha):
    N, cin, H, W = x_nchw.shape
    cc = weight.shape[0]
    s = 2
    cout = cc // (s * s)

    th = 64
    n_tiles = H // th

    w6 = weight.reshape(cout, s, s, cin, 3, 3)
    w2 = (jnp.transpose(w6, (4, 5, 3, 1, 2, 0))
          .reshape(9 * cin, cc).astype(jnp.bfloat16))
    b2 = (jnp.transpose(bias.reshape(cout, s, s), (1, 2, 0))
          .reshape(1, cc).astype(jnp.float32))
    a1 = jnp.asarray(alpha, jnp.float32).reshape(1)

    return pl.pallas_call(
        _fused_kernel,
        out_shape=jax.ShapeDtypeStruct((N, cout, s * H, s * W), jnp.float32),
        grid=(N, n_tiles),
        in_specs=[
            pl.BlockSpec((1, cin, H, W), lambda n, t: (n, 0, 0, 0)),
            pl.BlockSpec((9 * cin, cc), lambda n, t: (0, 0)),
            pl.BlockSpec((1, cc), lambda n, t: (0, 0)),
            pl.BlockSpec(memory_space=pltpu.MemorySpace.SMEM),
        ],
        out_specs=pl.BlockSpec((1, cout, s * th, s * W),
                               lambda n, t: (n, 0, t, 0)),
        scratch_shapes=[
            pltpu.VMEM((H + 2, W + 2, cin), jnp.float32),
            pltpu.VMEM((s * th, s * W, cout), jnp.float32),
        ],
        compiler_params=pltpu.CompilerParams(
            dimension_semantics=("parallel", "arbitrary"),
            vmem_limit_bytes=64 * 1024 * 1024),
    )(x_nchw, w2, b2, a1)


# speedup vs baseline: 1.1294x; 1.1294x over previous
import jax
import jax.numpy as jnp
from jax.experimental import pallas as pl
from jax.experimental.pallas import tpu as pltpu


def _fused_kernel(x_ref, w_ref, b_ref, a_ref, o_ref, xs_ref, s_ref):
    t = pl.program_id(1)
    cout = o_ref.shape[1]
    th = o_ref.shape[2] // 2
    w_out = o_ref.shape[3] // 2
    hh = x_ref.shape[2]
    ck = xs_ref.shape[2]

    @pl.when(t == 0)
    def _fill_input_scratch():
        base = jnp.transpose(x_ref[0], (1, 2, 0))
        g0 = jnp.pad(base, ((0, 0), (1, 0), (0, 0)))[:, :w_out, :]
        g2 = jnp.pad(base, ((0, 0), (0, 1), (0, 0)))[:, 1:, :]
        lanes = jnp.concatenate(
            [g0, base, g2, jnp.zeros_like(base)], axis=-1)
        xs_ref[0:1, :, :] = jnp.zeros_like(xs_ref[0:1, :, :])
        xs_ref[hh + 1:hh + 2, :, :] = jnp.zeros_like(xs_ref[0:1, :, :])
        xs_ref[1:hh + 1, :, :] = lanes.astype(xs_ref.dtype)

    row0 = pl.multiple_of(t * th, th)

    slabs = [
        xs_ref[pl.ds(row0 + dy, th), :, :].reshape(th * w_out, ck)
        for dy in range(3)
    ]
    patch = jnp.concatenate(slabs, axis=-1)

    acc = jnp.dot(patch, w_ref[...],
                  preferred_element_type=jnp.float32)
    acc = acc + b_ref[0]
    alpha = a_ref[0]
    acc = jnp.where(acc >= 0.0, acc, alpha * acc)

    for i in range(2):
        for j in range(2):
            lane0 = i * 2 * cout + j * cout
            v = acc[:, lane0:lane0 + cout].reshape(th, w_out, cout)
            s_ref[pl.ds(i, th, 2), pl.ds(j, w_out, 2), :] = v

    o_ref[0] = jnp.transpose(s_ref[...], (2, 0, 1))


def kernel(x_nchw, weight, bias, alpha):
    N, cin, H, W = x_nchw.shape
    cc = weight.shape[0]
    s = 2
    cout = cc // (s * s)

    th = 32
    n_tiles = H // th

    w6 = weight.reshape(cout, s, s, cin, 3, 3)
    w3 = jnp.transpose(w6, (4, 5, 3, 1, 2, 0)).reshape(3, 3 * cin, cc)
    w3 = jnp.pad(w3, ((0, 0), (0, cin), (0, 0))).reshape(12 * cin, cc)
    w3 = w3.astype(jnp.bfloat16)
    b2 = (jnp.transpose(bias.reshape(cout, s, s), (1, 2, 0))
          .reshape(1, cc).astype(jnp.float32))
    a1 = jnp.asarray(alpha, jnp.float32).reshape(1)

    return pl.pallas_call(
        _fused_kernel,
        out_shape=jax.ShapeDtypeStruct((N, cout, s * H, s * W), jnp.float32),
        grid=(N, n_tiles),
        in_specs=[
            pl.BlockSpec((1, cin, H, W), lambda n, t: (n, 0, 0, 0)),
            pl.BlockSpec((12 * cin, cc), lambda n, t: (0, 0)),
            pl.BlockSpec((1, cc), lambda n, t: (0, 0)),
            pl.BlockSpec(memory_space=pltpu.MemorySpace.SMEM),
        ],
        out_specs=pl.BlockSpec((1, cout, s * th, s * W),
                               lambda n, t: (n, 0, t, 0)),
        scratch_shapes=[
            pltpu.VMEM((H + 2, W, 4 * cin), jnp.bfloat16),
            pltpu.VMEM((s * th, s * W, cout), jnp.float32),
        ],
        compiler_params=pltpu.CompilerParams(
            dimension_semantics=("parallel", "arbitrary"),
            vmem_limit_bytes=64 * 1024 * 1024),
    )(x_nchw, w3, b2, a1)
